# baseline (speedup 1.0000x reference)
import functools

import jax
import jax.numpy as jnp
from jax import lax
from jax.experimental import pallas as pl
from jax.experimental.pallas import tpu as pltpu
from jax.experimental.pallas import tpu_sc as plsc

N_NUM = 13
N_CAT = 26
VOCAB = 100000
D = 64
B = 4096
NC = 2
NS = 16
NW = NC * NS
GROUP = 104
NGROUP = B * N_CAT // GROUP
GPW = NGROUP // NW
NBUF = 8
NIT = GPW // NBUF
N_TOK = 1 + N_NUM + N_CAT
BB = 512


def _sc_gather(idx_hbm, table_hbm, out_hbm, idx_v, buf, *sems):
    gsems = sems[:NBUF]
    wsems = sems[NBUF:]
    wid = lax.axis_index("s") * NC + lax.axis_index("c")
    g0 = wid * GPW
    o0 = wid * GPW * GROUP

    pltpu.sync_copy(idx_hbm.at[pl.ds(g0, GPW)], idx_v)

    def drain(sem, dst):
        pltpu.make_async_copy(table_hbm.at[pl.ds(0, GROUP)], dst, sem).wait()

    for b in range(NBUF):
        pltpu.async_copy(table_hbm.at[idx_v.at[b]], buf.at[b], gsems[b])

    def body(it, carry):
        for b in range(NBUF):
            g = it * NBUF + b
            drain(gsems[b], buf.at[b])
            pltpu.async_copy(
                buf.at[b], out_hbm.at[pl.ds(o0 + g * GROUP, GROUP)], wsems[b])
        for b in range(NBUF):
            g_next = (it + 1) * NBUF + b
            drain(wsems[b], out_hbm.at[pl.ds(0, GROUP)])
            pltpu.async_copy(
                table_hbm.at[idx_v.at[g_next]], buf.at[b], gsems[b])
        return carry

    lax.fori_loop(0, NIT - 1, body, 0)

    for b in range(NBUF):
        g = (NIT - 1) * NBUF + b
        drain(gsems[b], buf.at[b])
        pltpu.async_copy(
            buf.at[b], out_hbm.at[pl.ds(o0 + g * GROUP, GROUP)], wsems[b])
    for b in range(NBUF):
        drain(wsems[b], out_hbm.at[pl.ds(0, GROUP)])


def _tc_assemble(xn_ref, w_ref, bias_ref, cls_ref, cat_ref, o_ref):
    o_ref[:, 0, :] = jnp.broadcast_to(cls_ref[0, :][None, :], (BB, D))
    xv = xn_ref[...]
    w = w_ref[...]
    bias = bias_ref[...]
    for j in range(N_NUM):
        o_ref[:, 1 + j, :] = (
            xv[:, j][:, None] * w[j, :][None, :] + bias[j, :][None, :])
    o_ref[:, 1 + N_NUM:, :] = cat_ref[...]


@jax.jit
def _tokenize(x_num, idx_g, num_weight, num_bias, table_flat, cls_2d):
    mesh = plsc.VectorSubcoreMesh(core_axis_name="c", subcore_axis_name="s")
    gather = functools.partial(
        pl.kernel,
        out_type=jax.ShapeDtypeStruct((B * N_CAT, D), jnp.float32),
        mesh=mesh,
        compiler_params=pltpu.CompilerParams(use_tc_tiling_on_sc=False),
        scratch_types=[
            pltpu.VMEM((GPW, GROUP), jnp.int32),
            pltpu.VMEM((NBUF, GROUP, D), jnp.float32),
        ] + [pltpu.SemaphoreType.DMA] * (2 * NBUF),
    )(_sc_gather)
    cat_flat = gather(idx_g, table_flat)

    out = pl.pallas_call(
        _tc_assemble,
        grid=(B // BB,),
        in_specs=[
            pl.BlockSpec((BB, N_NUM), lambda i: (i, 0)),
            pl.BlockSpec((N_NUM, D), lambda i: (0, 0)),
            pl.BlockSpec((N_NUM, D), lambda i: (0, 0)),
            pl.BlockSpec((1, D), lambda i: (0, 0)),
            pl.BlockSpec((BB, N_CAT, D), lambda i: (i, 0, 0)),
        ],
        out_specs=pl.BlockSpec((BB, N_TOK, D), lambda i: (i, 0, 0)),
        out_shape=jax.ShapeDtypeStruct((B, N_TOK, D), jnp.float32),
    )(x_num, num_weight, num_bias, cls_2d,
      cat_flat.reshape(B, N_CAT, D))
    return out


def kernel(x_num, x_cat, num_weight, num_bias, cat_tables, cls_token):
    fidx = x_cat.astype(jnp.int32) + (
        jnp.arange(N_CAT, dtype=jnp.int32) * VOCAB)[None, :]
    idx_g = fidx.reshape(NGROUP, GROUP)
    table_flat = cat_tables.reshape(N_CAT * VOCAB, D)
    cls_2d = cls_token.reshape(1, D)
    return _tokenize(x_num, idx_g, num_weight, num_bias, table_flat, cls_2d)
